# baseline (speedup 1.0000x reference)
import functools

import jax
import jax.numpy as jnp
from jax import lax
from jax.experimental import pallas as pl
from jax.experimental.pallas import tpu as pltpu
from jax.experimental.pallas import tpu_sc as plsc

N = 10000
E = 320000
D_IN = 128
D_HID = 256
NCLS = 10

NC = 2
NS = 16
NW = NC * NS
C = 256
CH_W = 40
EP = NW * CH_W * C
NPAD = 10112
DEG_SLOTS = 10240

_mesh = plsc.VectorSubcoreMesh(
    core_axis_name="c", subcore_axis_name="s", num_cores=NC, num_subcores=NS)


@functools.partial(
    pl.kernel,
    out_type=jax.ShapeDtypeStruct((8, DEG_SLOTS), jnp.float32),
    mesh=_mesh,
    scratch_types=[
        pltpu.VMEM((C,), jnp.int32),
        pltpu.VMEM((C,), jnp.int32),
        pltpu.VMEM((C,), jnp.float32),
        pltpu.VMEM_SHARED((DEG_SLOTS,), jnp.float32),
        pltpu.VMEM_SHARED((DEG_SLOTS,), jnp.float32),
    ],
)
def _deg_kernel(srcp, dstp, ones_hbm, z1, out,
                src_c, dst_c, ones_v, dego_sh, degi_sh):
    c = lax.axis_index("c")
    s = lax.axis_index("s")
    w = s * NC + c
    pltpu.sync_copy(z1.at[pl.ds(s * 640, 640)], dego_sh.at[pl.ds(s * 640, 640)])
    pltpu.sync_copy(z1.at[pl.ds(s * 640, 640)], degi_sh.at[pl.ds(s * 640, 640)])
    pltpu.sync_copy(ones_hbm, ones_v)
    plsc.subcore_barrier()

    def chunk(j, carry):
        r = w * CH_W + j
        pltpu.sync_copy(srcp.at[r], src_c)
        pltpu.sync_copy(dstp.at[r], dst_c)
        pltpu.sync_copy(ones_v, dego_sh.at[src_c], add=True)
        pltpu.sync_copy(ones_v, degi_sh.at[dst_c], add=True)
        return carry

    lax.fori_loop(0, CH_W, chunk, 0)
    plsc.subcore_barrier()
    pltpu.sync_copy(dego_sh.at[pl.ds(s * 640, 640)],
                    out.at[2 * c + 0, pl.ds(s * 640, 640)])
    pltpu.sync_copy(degi_sh.at[pl.ds(s * 640, 640)],
                    out.at[2 * c + 1, pl.ds(s * 640, 640)])


@functools.partial(
    pl.kernel,
    out_type=jax.ShapeDtypeStruct((NC, NPAD, D_IN), jnp.float32),
    mesh=_mesh,
    scratch_types=[
        pltpu.VMEM((C,), jnp.int32),
        pltpu.VMEM((C,), jnp.int32),
        pltpu.VMEM((C, D_IN), jnp.float32),
        pltpu.VMEM_SHARED((NPAD, D_IN), jnp.float32),
        pltpu.SemaphoreType.DMA,
    ],
)
def _agg_kernel(vals, srcp, dstp, z2, out, src_c, dst_c, rows_v, acc_sh, sem):
    c = lax.axis_index("c")
    s = lax.axis_index("s")
    w = s * NC + c
    pltpu.sync_copy(z2.at[pl.ds(s * 632, 632)], acc_sh.at[pl.ds(s * 632, 632)])
    plsc.subcore_barrier()

    def chunk(j, carry):
        r = w * CH_W + j
        pltpu.sync_copy(srcp.at[r], src_c)
        pltpu.sync_copy(dstp.at[r], dst_c)
        pltpu.async_copy(vals.at[src_c], rows_v, sem).wait()
        pltpu.sync_copy(rows_v, acc_sh.at[dst_c], add=True)
        return carry

    lax.fori_loop(0, CH_W, chunk, 0)
    plsc.subcore_barrier()
    pltpu.sync_copy(acc_sh.at[pl.ds(s * 632, 632)],
                    out.at[c, pl.ds(s * 632, 632)])


def _tc1_body(degp_ref, x_ref, xp_ref, ns_ref, nd_ref):
    d_o = degp_ref[0] + degp_ref[2]
    d_i = degp_ref[1] + degp_ref[3]
    ns = jnp.where(d_o > 0, lax.rsqrt(d_o), 0.0)
    nd = jnp.where(d_i > 0, lax.rsqrt(d_i), 0.0)
    ns_c = jnp.reshape(ns, (DEG_SLOTS, 1))[:N]
    nd_c = jnp.reshape(nd, (DEG_SLOTS, 1))[:N]
    ns_ref[...] = ns_c
    nd_ref[...] = nd_c
    xp_ref[...] = jnp.concatenate(
        [x_ref[...] * ns_c, jnp.zeros((NPAD - N, D_IN), jnp.float32)], axis=0)


def _tc2_body(aggp_ref, ns_ref, nd_ref, w1_ref, b1_ref, h1a_ref, h1b_ref):
    agg = (aggp_ref[0] + aggp_ref[1])[:N]
    t = jnp.dot(agg, w1_ref[...], preferred_element_type=jnp.float32)
    h = t * nd_ref[...] + b1_ref[...][None, :]
    h = jnp.where(h >= 0, h, 0.01 * h)
    hp = h * ns_ref[...]
    z = jnp.zeros((NPAD - N, D_IN), jnp.float32)
    h1a_ref[...] = jnp.concatenate([hp[:, :D_IN], z], axis=0)
    h1b_ref[...] = jnp.concatenate([hp[:, D_IN:], z], axis=0)


def _tc3_body(p2a_ref, p2b_ref, nd_ref, w2_ref, b2_ref, wc_ref, bc_ref, out_ref):
    agg = jnp.concatenate(
        [(p2a_ref[0] + p2a_ref[1])[:N], (p2b_ref[0] + p2b_ref[1])[:N]], axis=1)
    t = jnp.dot(agg, w2_ref[...], preferred_element_type=jnp.float32)
    h = t * nd_ref[...] + b2_ref[...][None, :]
    h = jnp.where(h >= 0, h, 0.01 * h)
    hg = jnp.sum(h, axis=0, keepdims=True) * (1.0 / N)
    out_ref[...] = (jnp.dot(hg, wc_ref[...], preferred_element_type=jnp.float32)
                    + bc_ref[...][None, :])


_tc1 = pl.pallas_call(
    _tc1_body,
    out_shape=[jax.ShapeDtypeStruct((NPAD, D_IN), jnp.float32),
               jax.ShapeDtypeStruct((N, 1), jnp.float32),
               jax.ShapeDtypeStruct((N, 1), jnp.float32)])

_tc2 = pl.pallas_call(
    _tc2_body,
    out_shape=[jax.ShapeDtypeStruct((NPAD, D_IN), jnp.float32),
               jax.ShapeDtypeStruct((NPAD, D_IN), jnp.float32)])

_tc3 = pl.pallas_call(
    _tc3_body,
    out_shape=jax.ShapeDtypeStruct((1, NCLS), jnp.float32))


@jax.jit
def _run(x, edge_index, W1, b1, W2, b2, Wc, bc):
    src = edge_index[0]
    dst = edge_index[1]
    pad = jnp.full((EP - E,), N, jnp.int32)
    srcp = jnp.concatenate([src, pad]).reshape(EP // C, C)
    dstp = jnp.concatenate([dst, pad]).reshape(EP // C, C)

    ones_c = jnp.ones((C,), jnp.float32)
    z1 = jnp.zeros((DEG_SLOTS,), jnp.float32)
    z2 = jnp.zeros((NPAD, D_IN), jnp.float32)

    degp = _deg_kernel(srcp, dstp, ones_c, z1)
    xp, ns, nd = _tc1(degp, x)
    aggx = _agg_kernel(xp, srcp, dstp, z2)
    h1a, h1b = _tc2(aggx, ns, nd, W1, b1)
    p2a = _agg_kernel(h1a, srcp, dstp, z2)
    p2b = _agg_kernel(h1b, srcp, dstp, z2)
    out = _tc3(p2a, p2b, nd, W2, b2, Wc, bc)
    return out.reshape(NCLS)


def kernel(x, edge_index, W1, b1, W2, b2, Wc, bc):
    return _run(x, edge_index, W1, b1, W2, b2, Wc, bc)


# speedup vs baseline: 1.3210x; 1.3210x over previous
import functools

import jax
import jax.numpy as jnp
from jax import lax
from jax.experimental import pallas as pl
from jax.experimental.pallas import tpu as pltpu
from jax.experimental.pallas import tpu_sc as plsc

N = 10000
E = 320000
D_IN = 128
D_HID = 256
NCLS = 10

NC = 2
NS = 16
NW = NC * NS
C = 128
CH_W = 79
EP = NW * CH_W * C
NPAD = 10112
DEG_SLOTS = 10240

_mesh = plsc.VectorSubcoreMesh(
    core_axis_name="c", subcore_axis_name="s", num_cores=NC, num_subcores=NS)


@functools.partial(
    pl.kernel,
    out_type=jax.ShapeDtypeStruct((8, DEG_SLOTS), jnp.float32),
    mesh=_mesh,
    scratch_types=[
        pltpu.VMEM((C,), jnp.int32),
        pltpu.VMEM((C,), jnp.int32),
        pltpu.VMEM((C,), jnp.float32),
        pltpu.VMEM_SHARED((DEG_SLOTS,), jnp.float32),
        pltpu.VMEM_SHARED((DEG_SLOTS,), jnp.float32),
    ],
)
def _deg_kernel(srcp, dstp, ones_hbm, z1, out,
                src_c, dst_c, ones_v, dego_sh, degi_sh):
    c = lax.axis_index("c")
    s = lax.axis_index("s")
    w = s * NC + c
    pltpu.sync_copy(z1.at[pl.ds(s * 640, 640)], dego_sh.at[pl.ds(s * 640, 640)])
    pltpu.sync_copy(z1.at[pl.ds(s * 640, 640)], degi_sh.at[pl.ds(s * 640, 640)])
    pltpu.sync_copy(ones_hbm, ones_v)
    plsc.subcore_barrier()

    def chunk(j, carry):
        r = w * CH_W + j
        pltpu.sync_copy(srcp.at[r], src_c)
        pltpu.sync_copy(dstp.at[r], dst_c)
        pltpu.sync_copy(ones_v, dego_sh.at[src_c], add=True)
        pltpu.sync_copy(ones_v, degi_sh.at[dst_c], add=True)
        return carry

    lax.fori_loop(0, CH_W, chunk, 0)
    plsc.subcore_barrier()
    pltpu.sync_copy(dego_sh.at[pl.ds(s * 640, 640)],
                    out.at[2 * c + 0, pl.ds(s * 640, 640)])
    pltpu.sync_copy(degi_sh.at[pl.ds(s * 640, 640)],
                    out.at[2 * c + 1, pl.ds(s * 640, 640)])


@functools.partial(
    pl.kernel,
    out_type=jax.ShapeDtypeStruct((NC, NPAD, D_IN), jnp.float32),
    mesh=_mesh,
    scratch_types=[
        pltpu.VMEM((C,), jnp.int32),
        pltpu.VMEM((C,), jnp.int32),
        pltpu.VMEM((C, D_IN), jnp.float32),
        pltpu.VMEM_SHARED((NPAD, D_IN), jnp.float32),
        pltpu.SemaphoreType.DMA,
    ],
)
def _agg_kernel(vals, srcp, dstp, z2, out, src_c, dst_c, rows_v, acc_sh, sem):
    c = lax.axis_index("c")
    s = lax.axis_index("s")
    w = s * NC + c
    pltpu.sync_copy(z2.at[pl.ds(s * 632, 632)], acc_sh.at[pl.ds(s * 632, 632)])
    plsc.subcore_barrier()

    def chunk(j, carry):
        r = w * CH_W + j
        pltpu.sync_copy(srcp.at[r], src_c)
        pltpu.sync_copy(dstp.at[r], dst_c)
        pltpu.async_copy(vals.at[src_c], rows_v, sem).wait()
        pltpu.sync_copy(rows_v, acc_sh.at[dst_c], add=True)
        return carry

    lax.fori_loop(0, CH_W, chunk, 0)
    plsc.subcore_barrier()
    pltpu.sync_copy(acc_sh.at[pl.ds(s * 632, 632)],
                    out.at[c, pl.ds(s * 632, 632)])


def _tc1_body(degp_ref, x_ref, xp_ref, ns_ref, nd_ref):
    d_o = degp_ref[0] + degp_ref[2]
    d_i = degp_ref[1] + degp_ref[3]
    ns = jnp.where(d_o > 0, lax.rsqrt(d_o), 0.0)
    nd = jnp.where(d_i > 0, lax.rsqrt(d_i), 0.0)
    ns_c = jnp.reshape(ns, (DEG_SLOTS, 1))[:N]
    nd_c = jnp.reshape(nd, (DEG_SLOTS, 1))[:N]
    ns_ref[...] = ns_c
    nd_ref[...] = nd_c
    xp_ref[...] = jnp.concatenate(
        [x_ref[...] * ns_c, jnp.zeros((NPAD - N, D_IN), jnp.float32)], axis=0)


def _tc2_body(aggp_ref, ns_ref, nd_ref, w1_ref, b1_ref, h1a_ref, h1b_ref):
    agg = (aggp_ref[0] + aggp_ref[1])[:N]
    t = jnp.dot(agg, w1_ref[...], preferred_element_type=jnp.float32)
    h = t * nd_ref[...] + b1_ref[...][None, :]
    h = jnp.where(h >= 0, h, 0.01 * h)
    hp = h * ns_ref[...]
    z = jnp.zeros((NPAD - N, D_IN), jnp.float32)
    h1a_ref[...] = jnp.concatenate([hp[:, :D_IN], z], axis=0)
    h1b_ref[...] = jnp.concatenate([hp[:, D_IN:], z], axis=0)


def _tc3_body(p2a_ref, p2b_ref, nd_ref, w2_ref, b2_ref, wc_ref, bc_ref, out_ref):
    agg = jnp.concatenate(
        [(p2a_ref[0] + p2a_ref[1])[:N], (p2b_ref[0] + p2b_ref[1])[:N]], axis=1)
    t = jnp.dot(agg, w2_ref[...], preferred_element_type=jnp.float32)
    h = t * nd_ref[...] + b2_ref[...][None, :]
    h = jnp.where(h >= 0, h, 0.01 * h)
    hg = jnp.sum(h, axis=0, keepdims=True) * (1.0 / N)
    out_ref[...] = (jnp.dot(hg, wc_ref[...], preferred_element_type=jnp.float32)
                    + bc_ref[...][None, :])


_tc1 = pl.pallas_call(
    _tc1_body,
    out_shape=[jax.ShapeDtypeStruct((NPAD, D_IN), jnp.float32),
               jax.ShapeDtypeStruct((N, 1), jnp.float32),
               jax.ShapeDtypeStruct((N, 1), jnp.float32)])

_tc2 = pl.pallas_call(
    _tc2_body,
    out_shape=[jax.ShapeDtypeStruct((NPAD, D_IN), jnp.float32),
               jax.ShapeDtypeStruct((NPAD, D_IN), jnp.float32)])

_tc3 = pl.pallas_call(
    _tc3_body,
    out_shape=jax.ShapeDtypeStruct((1, NCLS), jnp.float32))


@jax.jit
def _run(x, edge_index, W1, b1, W2, b2, Wc, bc):
    src = edge_index[0]
    dst = edge_index[1]
    pad = jnp.full((EP - E,), N, jnp.int32)
    srcp = jnp.concatenate([src, pad]).reshape(EP // C, C)
    dstp = jnp.concatenate([dst, pad]).reshape(EP // C, C)

    ones_c = jnp.ones((C,), jnp.float32)
    z1 = jnp.zeros((DEG_SLOTS,), jnp.float32)
    z2 = jnp.zeros((NPAD, D_IN), jnp.float32)

    degp = _deg_kernel(srcp, dstp, ones_c, z1)
    xp, ns, nd = _tc1(degp, x)
    aggx = _agg_kernel(xp, srcp, dstp, z2)
    h1a, h1b = _tc2(aggx, ns, nd, W1, b1)
    p2a = _agg_kernel(h1a, srcp, dstp, z2)
    p2b = _agg_kernel(h1b, srcp, dstp, z2)
    out = _tc3(p2a, p2b, nd, W2, b2, Wc, bc)
    return out.reshape(NCLS)


def kernel(x, edge_index, W1, b1, W2, b2, Wc, bc):
    return _run(x, edge_index, W1, b1, W2, b2, Wc, bc)


# speedup vs baseline: 1.9866x; 1.5038x over previous
import functools

import jax
import jax.numpy as jnp
from jax import lax
from jax.experimental import pallas as pl
from jax.experimental.pallas import tpu as pltpu
from jax.experimental.pallas import tpu_sc as plsc

N = 10000
E = 320000
D_IN = 128
D_HID = 256
NCLS = 10

NC = 2
NS = 16
NW = NC * NS
C = 128
CH_W = 79
EP = NW * CH_W * C
NPAD = 10112
DEG_SLOTS = 10240

_mesh = plsc.VectorSubcoreMesh(
    core_axis_name="c", subcore_axis_name="s", num_cores=NC, num_subcores=NS)


@functools.partial(
    pl.kernel,
    out_type=jax.ShapeDtypeStruct((8, DEG_SLOTS), jnp.float32),
    mesh=_mesh,
    scratch_types=[
        pltpu.VMEM((C,), jnp.int32),
        pltpu.VMEM((C,), jnp.int32),
        pltpu.VMEM((C,), jnp.float32),
        pltpu.VMEM_SHARED((DEG_SLOTS,), jnp.float32),
        pltpu.VMEM_SHARED((DEG_SLOTS,), jnp.float32),
    ],
)
def _deg_kernel(srcp, dstp, ones_hbm, z1, out,
                src_c, dst_c, ones_v, dego_sh, degi_sh):
    c = lax.axis_index("c")
    s = lax.axis_index("s")
    w = s * NC + c
    pltpu.sync_copy(z1.at[pl.ds(s * 640, 640)], dego_sh.at[pl.ds(s * 640, 640)])
    pltpu.sync_copy(z1.at[pl.ds(s * 640, 640)], degi_sh.at[pl.ds(s * 640, 640)])
    pltpu.sync_copy(ones_hbm, ones_v)
    plsc.subcore_barrier()

    def chunk(j, carry):
        r = w * CH_W + j
        pltpu.sync_copy(srcp.at[r], src_c)
        pltpu.sync_copy(dstp.at[r], dst_c)
        pltpu.sync_copy(ones_v, dego_sh.at[src_c], add=True)
        pltpu.sync_copy(ones_v, degi_sh.at[dst_c], add=True)
        return carry

    lax.fori_loop(0, CH_W, chunk, 0)
    plsc.subcore_barrier()
    pltpu.sync_copy(dego_sh.at[pl.ds(s * 640, 640)],
                    out.at[2 * c + 0, pl.ds(s * 640, 640)])
    pltpu.sync_copy(degi_sh.at[pl.ds(s * 640, 640)],
                    out.at[2 * c + 1, pl.ds(s * 640, 640)])


@functools.partial(
    pl.kernel,
    out_type=jax.ShapeDtypeStruct((NC, NPAD, D_IN), jnp.float32),
    mesh=_mesh,
    scratch_types=[
        pltpu.VMEM((C,), jnp.int32),
        pltpu.VMEM((C,), jnp.int32),
        pltpu.VMEM((C, D_IN), jnp.float32),
        pltpu.VMEM_SHARED((NPAD, D_IN), jnp.float32),
        pltpu.SemaphoreType.DMA,
    ],
)
def _agg_kernel(vals, srcp, dstp, z2, out, src_c, dst_c, rows_v, acc_sh, sem):
    c = lax.axis_index("c")
    s = lax.axis_index("s")
    w = s * NC + c
    pltpu.sync_copy(z2.at[pl.ds(s * 632, 632)], acc_sh.at[pl.ds(s * 632, 632)])
    plsc.subcore_barrier()

    def chunk(j, carry):
        r = w * CH_W + j
        pltpu.sync_copy(srcp.at[r], src_c)
        pltpu.sync_copy(dstp.at[r], dst_c)
        pltpu.async_copy(vals.at[src_c], rows_v, sem).wait()
        pltpu.sync_copy(rows_v, acc_sh.at[dst_c], add=True)
        return carry

    lax.fori_loop(0, CH_W, chunk, 0)
    plsc.subcore_barrier()
    pltpu.sync_copy(acc_sh.at[pl.ds(s * 632, 632)],
                    out.at[c, pl.ds(s * 632, 632)])


def _tc1_body(degp_ref, x_ref, xp_ref, ns_ref, nd_ref):
    d_o = degp_ref[0] + degp_ref[2]
    d_i = degp_ref[1] + degp_ref[3]
    ns = jnp.where(d_o > 0, lax.rsqrt(d_o), 0.0)
    nd = jnp.where(d_i > 0, lax.rsqrt(d_i), 0.0)
    ns_c = jnp.reshape(ns, (DEG_SLOTS, 1))[:N]
    nd_c = jnp.reshape(nd, (DEG_SLOTS, 1))[:N]
    ns_ref[...] = ns_c
    nd_ref[...] = nd_c
    xp_ref[...] = jnp.concatenate(
        [x_ref[...] * ns_c, jnp.zeros((NPAD - N, D_IN), jnp.float32)], axis=0)


def _tc2_body(aggp_ref, ns_ref, nd_ref, w1_ref, b1_ref, h1a_ref, h1b_ref):
    agg = (aggp_ref[0] + aggp_ref[1])[:N]
    t = jnp.dot(agg, w1_ref[...], preferred_element_type=jnp.float32)
    h = t * nd_ref[...] + b1_ref[...][None, :]
    h = jnp.where(h >= 0, h, 0.01 * h)
    hp = h * ns_ref[...]
    z = jnp.zeros((NPAD - N, D_IN), jnp.float32)
    h1a_ref[...] = jnp.concatenate([hp[:, :D_IN], z], axis=0)
    h1b_ref[...] = jnp.concatenate([hp[:, D_IN:], z], axis=0)


def _tc3_body(p2a_ref, p2b_ref, nd_ref, w2_ref, b2_ref, wc_ref, bc_ref, out_ref):
    agg = jnp.concatenate(
        [(p2a_ref[0] + p2a_ref[1])[:N], (p2b_ref[0] + p2b_ref[1])[:N]], axis=1)
    t = jnp.dot(agg, w2_ref[...], preferred_element_type=jnp.float32)
    h = t * nd_ref[...] + b2_ref[...][None, :]
    h = jnp.where(h >= 0, h, 0.01 * h)
    hg = jnp.sum(h, axis=0, keepdims=True) * (1.0 / N)
    out_ref[...] = (jnp.dot(hg, wc_ref[...], preferred_element_type=jnp.float32)
                    + bc_ref[...][None, :])


_tc1 = pl.pallas_call(
    _tc1_body,
    out_shape=[jax.ShapeDtypeStruct((NPAD, D_IN), jnp.float32),
               jax.ShapeDtypeStruct((N, 1), jnp.float32),
               jax.ShapeDtypeStruct((N, 1), jnp.float32)])

_tc2 = pl.pallas_call(
    _tc2_body,
    out_shape=[jax.ShapeDtypeStruct((NPAD, D_IN), jnp.float32),
               jax.ShapeDtypeStruct((NPAD, D_IN), jnp.float32)])

_tc3 = pl.pallas_call(
    _tc3_body,
    out_shape=jax.ShapeDtypeStruct((1, NCLS), jnp.float32))


@jax.jit
def _run(x, edge_index, W1, b1, W2, b2, Wc, bc):
    src = edge_index[0]
    dst = edge_index[1]
    pad = N + (jnp.arange(EP - E, dtype=jnp.int32) % (NPAD - N))
    srcp = jnp.concatenate([src, pad]).reshape(EP // C, C)
    dstp = jnp.concatenate([dst, pad]).reshape(EP // C, C)

    ones_c = jnp.ones((C,), jnp.float32)
    z1 = jnp.zeros((DEG_SLOTS,), jnp.float32)
    z2 = jnp.zeros((NPAD, D_IN), jnp.float32)

    degp = _deg_kernel(srcp, dstp, ones_c, z1)
    xp, ns, nd = _tc1(degp, x)
    aggx = _agg_kernel(xp, srcp, dstp, z2)
    h1a, h1b = _tc2(aggx, ns, nd, W1, b1)
    p2a = _agg_kernel(h1a, srcp, dstp, z2)
    p2b = _agg_kernel(h1b, srcp, dstp, z2)
    out = _tc3(p2a, p2b, nd, W2, b2, Wc, bc)
    return out.reshape(NCLS)


def kernel(x, edge_index, W1, b1, W2, b2, Wc, bc):
    return _run(x, edge_index, W1, b1, W2, b2, Wc, bc)


# speedup vs baseline: 3.6255x; 1.8250x over previous
import functools

import jax
import jax.numpy as jnp
from jax import lax
from jax.experimental import pallas as pl
from jax.experimental.pallas import tpu as pltpu
from jax.experimental.pallas import tpu_sc as plsc

N = 10000
E = 320000
D_IN = 128
D_HID = 256
NCLS = 10

NC = 2
NS = 16
NW = NC * NS
C = 128
CH_W = 80
EP = NW * CH_W * C
NPAD = 10112
DEG_SLOTS = 10240

_mesh = plsc.VectorSubcoreMesh(
    core_axis_name="c", subcore_axis_name="s", num_cores=NC, num_subcores=NS)


def _unpack(pk_v, j, src_c, dst_c):
    for g in range(C // 16):
        v = pk_v[j, pl.ds(g * 16, 16)]
        src_c[pl.ds(g * 16, 16)] = v >> 14
        dst_c[pl.ds(g * 16, 16)] = v & 16383


@functools.partial(
    pl.kernel,
    out_type=jax.ShapeDtypeStruct((8, DEG_SLOTS), jnp.float32),
    mesh=_mesh,
    scratch_types=[
        pltpu.VMEM((CH_W, C), jnp.int32),
        pltpu.VMEM((C,), jnp.int32),
        pltpu.VMEM((C,), jnp.int32),
        pltpu.VMEM((C,), jnp.float32),
        pltpu.VMEM_SHARED((DEG_SLOTS,), jnp.float32),
        pltpu.VMEM_SHARED((DEG_SLOTS,), jnp.float32),
    ],
)
def _deg_kernel(pkp, ones_hbm, z1, out,
                pk_v, src_c, dst_c, ones_v, dego_sh, degi_sh):
    c = lax.axis_index("c")
    s = lax.axis_index("s")
    w = s * NC + c
    pltpu.sync_copy(z1.at[pl.ds(s * 640, 640)], dego_sh.at[pl.ds(s * 640, 640)])
    pltpu.sync_copy(z1.at[pl.ds(s * 640, 640)], degi_sh.at[pl.ds(s * 640, 640)])
    pltpu.sync_copy(ones_hbm, ones_v)
    pltpu.sync_copy(pkp.at[pl.ds(w * CH_W, CH_W)], pk_v)
    plsc.subcore_barrier()

    def chunk(j, carry):
        _unpack(pk_v, j, src_c, dst_c)
        pltpu.sync_copy(ones_v, dego_sh.at[src_c], add=True)
        pltpu.sync_copy(ones_v, degi_sh.at[dst_c], add=True)
        return carry

    lax.fori_loop(0, CH_W, chunk, 0)
    plsc.subcore_barrier()
    pltpu.sync_copy(dego_sh.at[pl.ds(s * 640, 640)],
                    out.at[2 * c + 0, pl.ds(s * 640, 640)])
    pltpu.sync_copy(degi_sh.at[pl.ds(s * 640, 640)],
                    out.at[2 * c + 1, pl.ds(s * 640, 640)])


@functools.partial(
    pl.kernel,
    out_type=jax.ShapeDtypeStruct((NC, NPAD, D_IN), jnp.float32),
    mesh=_mesh,
    scratch_types=[
        pltpu.VMEM((CH_W, C), jnp.int32),
        pltpu.VMEM((C,), jnp.int32),
        pltpu.VMEM((C,), jnp.int32),
        pltpu.VMEM((C,), jnp.int32),
        pltpu.VMEM((C,), jnp.int32),
        pltpu.VMEM((C, D_IN), jnp.float32),
        pltpu.VMEM((C, D_IN), jnp.float32),
        pltpu.VMEM_SHARED((NPAD, D_IN), jnp.float32),
        pltpu.SemaphoreType.DMA,
        pltpu.SemaphoreType.DMA,
        pltpu.SemaphoreType.DMA,
        pltpu.SemaphoreType.DMA,
    ],
)
def _agg_kernel(vals, pkp, z2, out,
                pk_v, src0, dst0, src1, dst1, rows0, rows1, acc_sh,
                g0, g1, s0, s1):
    c = lax.axis_index("c")
    s = lax.axis_index("s")
    w = s * NC + c
    ring = ((src0, dst0, rows0, g0, s0), (src1, dst1, rows1, g1, s1))

    pltpu.sync_copy(z2.at[pl.ds(s * 632, 632)], acc_sh.at[pl.ds(s * 632, 632)])
    pltpu.sync_copy(pkp.at[pl.ds(w * CH_W, CH_W)], pk_v)
    plsc.subcore_barrier()

    _unpack(pk_v, 0, src0, dst0)
    pltpu.async_copy(vals.at[src0], rows0, g0)

    def pair(jo, carry):
        for b in range(2):
            j = 2 * jo + b
            srcb, dstb, rowsb, gb, sb = ring[b]
            srco, dsto, rowso, go, so = ring[1 - b]
            pltpu.make_async_copy(vals.at[srcb], rowsb, gb).wait()
            pltpu.async_copy(rowsb, acc_sh.at[dstb], sb, add=True)

            @pl.when(j + 1 < CH_W)
            def _():
                @pl.when(j >= 1)
                def _():
                    pltpu.make_async_copy(
                        rowso, acc_sh.at[dsto], so).wait()
                _unpack(pk_v, j + 1, srco, dsto)
                pltpu.async_copy(vals.at[srco], rowso, go)
        return carry

    lax.fori_loop(0, CH_W // 2, pair, 0)
    pltpu.make_async_copy(rows0, acc_sh.at[dst0], s0).wait()
    pltpu.make_async_copy(rows1, acc_sh.at[dst1], s1).wait()

    plsc.subcore_barrier()
    pltpu.sync_copy(acc_sh.at[pl.ds(s * 632, 632)],
                    out.at[c, pl.ds(s * 632, 632)])


def _tc1_body(degp_ref, x_ref, xp_ref, ns_ref, nd_ref):
    d_o = degp_ref[0] + degp_ref[2]
    d_i = degp_ref[1] + degp_ref[3]
    ns = jnp.where(d_o > 0, lax.rsqrt(d_o), 0.0)
    nd = jnp.where(d_i > 0, lax.rsqrt(d_i), 0.0)
    ns_c = jnp.reshape(ns, (DEG_SLOTS, 1))[:N]
    nd_c = jnp.reshape(nd, (DEG_SLOTS, 1))[:N]
    ns_ref[...] = ns_c
    nd_ref[...] = nd_c
    xp_ref[...] = jnp.concatenate(
        [x_ref[...] * ns_c, jnp.zeros((NPAD - N, D_IN), jnp.float32)], axis=0)


def _tc2_body(aggp_ref, ns_ref, nd_ref, w1_ref, b1_ref, h1a_ref, h1b_ref):
    agg = (aggp_ref[0] + aggp_ref[1])[:N]
    t = jnp.dot(agg, w1_ref[...], preferred_element_type=jnp.float32)
    h = t * nd_ref[...] + b1_ref[...][None, :]
    h = jnp.where(h >= 0, h, 0.01 * h)
    hp = h * ns_ref[...]
    z = jnp.zeros((NPAD - N, D_IN), jnp.float32)
    h1a_ref[...] = jnp.concatenate([hp[:, :D_IN], z], axis=0)
    h1b_ref[...] = jnp.concatenate([hp[:, D_IN:], z], axis=0)


def _tc3_body(p2a_ref, p2b_ref, nd_ref, w2_ref, b2_ref, wc_ref, bc_ref, out_ref):
    agg = jnp.concatenate(
        [(p2a_ref[0] + p2a_ref[1])[:N], (p2b_ref[0] + p2b_ref[1])[:N]], axis=1)
    t = jnp.dot(agg, w2_ref[...], preferred_element_type=jnp.float32)
    h = t * nd_ref[...] + b2_ref[...][None, :]
    h = jnp.where(h >= 0, h, 0.01 * h)
    hg = jnp.sum(h, axis=0, keepdims=True) * (1.0 / N)
    out_ref[...] = (jnp.dot(hg, wc_ref[...], preferred_element_type=jnp.float32)
                    + bc_ref[...][None, :])


_tc1 = pl.pallas_call(
    _tc1_body,
    out_shape=[jax.ShapeDtypeStruct((NPAD, D_IN), jnp.float32),
               jax.ShapeDtypeStruct((N, 1), jnp.float32),
               jax.ShapeDtypeStruct((N, 1), jnp.float32)])

_tc2 = pl.pallas_call(
    _tc2_body,
    out_shape=[jax.ShapeDtypeStruct((NPAD, D_IN), jnp.float32),
               jax.ShapeDtypeStruct((NPAD, D_IN), jnp.float32)])

_tc3 = pl.pallas_call(
    _tc3_body,
    out_shape=jax.ShapeDtypeStruct((1, NCLS), jnp.float32))


@jax.jit
def _run(x, edge_index, W1, b1, W2, b2, Wc, bc):
    src = edge_index[0]
    dst = edge_index[1]
    pad = N + (jnp.arange(EP - E, dtype=jnp.int32) % (NPAD - N))
    srcp = jnp.concatenate([src, pad])
    dstp = jnp.concatenate([dst, pad])
    pkp = ((srcp << 14) | dstp).reshape(EP // C, C)

    ones_c = jnp.ones((C,), jnp.float32)
    z1 = jnp.zeros((DEG_SLOTS,), jnp.float32)
    z2 = jnp.zeros((NPAD, D_IN), jnp.float32)

    degp = _deg_kernel(pkp, ones_c, z1)
    xp, ns, nd = _tc1(degp, x)
    aggx = _agg_kernel(xp, pkp, z2)
    h1a, h1b = _tc2(aggx, ns, nd, W1, b1)
    p2a = _agg_kernel(h1a, pkp, z2)
    p2b = _agg_kernel(h1b, pkp, z2)
    out = _tc3(p2a, p2b, nd, W2, b2, Wc, bc)
    return out.reshape(NCLS)


def kernel(x, edge_index, W1, b1, W2, b2, Wc, bc):
    return _run(x, edge_index, W1, b1, W2, b2, Wc, bc)
